# baseline (speedup 1.0000x reference)
import functools

import jax
import jax.numpy as jnp
from jax import lax
from jax.experimental import pallas as pl
from jax.experimental.pallas import tpu as pltpu
from jax.experimental.pallas import tpu_sc as plsc

N = 10000
NP = 10240
H = 128
E = 320000
B = 512
T = 4

NC = 2
NS = 16
NW = NC * NS
C = 80
PCH = 64
NPASS = 2
EWP = NPASS * PCH * C
EP = NW * EWP
RT = NP // NS
ZC = RT // C
BW = B // NW

_sc_mesh = plsc.VectorSubcoreMesh(
    core_axis_name="c", subcore_axis_name="s", num_cores=NC, num_subcores=NS)


@functools.partial(
    pl.kernel,
    out_type=jax.ShapeDtypeStruct((NC, NP, H), jnp.float32),
    mesh=_sc_mesh,
    scratch_types=[
        pltpu.VMEM((PCH, C), jnp.int32),
        pltpu.VMEM((PCH, C), jnp.int32),
        pltpu.VMEM((C, H), jnp.float32),
        pltpu.VMEM((C, H), jnp.float32),
        pltpu.VMEM_SHARED((NP, H), jnp.float32),
        pltpu.SemaphoreType.DMA,
        pltpu.SemaphoreType.DMA,
    ],
)
def _sc_segsum(mw_hbm, src_hbm, dst_hbm, zero_hbm, out_hbm,
               sidx, didx, rows0, rows1, acc, sem0, sem1):
    cid = lax.axis_index("c")
    sid = lax.axis_index("s")
    wid = sid * NC + cid
    pltpu.sync_copy(zero_hbm, rows0)
    for k in range(ZC):
        pltpu.sync_copy(rows0, acc.at[pl.ds(sid * RT + k * C, C)])
    plsc.subcore_barrier()

    for p in range(NPASS):
        pltpu.sync_copy(src_hbm.at[wid, p], sidx)
        pltpu.sync_copy(dst_hbm.at[wid, p], didx)
        @pl.loop(0, PCH)
        def _chunk(j):
            pltpu.async_copy(mw_hbm.at[sidx.at[j]], rows0, sem0).wait()
            pltpu.sync_copy(rows0, acc.at[didx.at[j]], add=True)

    plsc.subcore_barrier()
    for k in range(ZC):
        pltpu.sync_copy(acc.at[pl.ds(sid * RT + k * C, C)], rows0)
        pltpu.sync_copy(rows0, out_hbm.at[cid, pl.ds(sid * RT + k * C, C)])


@functools.partial(
    pl.kernel,
    out_type=jax.ShapeDtypeStruct((B, H), jnp.float32),
    mesh=_sc_mesh,
    scratch_types=[
        pltpu.VMEM((BW,), jnp.int32),
        pltpu.VMEM((BW, H), jnp.float32),
        pltpu.SemaphoreType.DMA,
    ],
)
def _sc_gather(z_hbm, idx_hbm, out_hbm, idx_v, rows_v, sem):
    wid = lax.axis_index("s") * NC + lax.axis_index("c")
    base = wid * BW
    pltpu.sync_copy(idx_hbm.at[pl.ds(base, BW)], idx_v)
    pltpu.async_copy(z_hbm.at[idx_v], rows_v, sem).wait()
    pltpu.sync_copy(rows_v, out_hbm.at[pl.ds(base, BW)])


def _bf(x):
    return x.astype(jnp.bfloat16)


def _gru_math(p_ref, h_ref, Wi, Wh, bi, bh):
    h = h_ref[...]
    agg = p_ref[0] + p_ref[1]
    gi = jnp.dot(_bf(agg), Wi[...],
                 preferred_element_type=jnp.float32) + bi[...]
    gh = jnp.dot(_bf(h), Wh[...],
                 preferred_element_type=jnp.float32) + bh[...]
    r = jax.nn.sigmoid(gi[:, :H] + gh[:, :H])
    z = jax.nn.sigmoid(gi[:, H:2 * H] + gh[:, H:2 * H])
    n = jnp.tanh(gi[:, 2 * H:] + r * gh[:, 2 * H:])
    return (1.0 - z) * n + z * h


def _gru_body(p_ref, h_ref, We, Wi, Wh, bi, bh, h_out, mw_out):
    h_new = _gru_math(p_ref, h_ref, Wi, Wh, bi, bh)
    h_out[...] = h_new
    mw_out[...] = jnp.dot(_bf(h_new), We[...],
                          preferred_element_type=jnp.float32)


def _gru_final_body(p_ref, h_ref, Wi, Wh, bi, bh, Wf, bf_, out_ref):
    h_new = _gru_math(p_ref, h_ref, Wi, Wh, bi, bh)
    out_ref[...] = jnp.dot(_bf(h_new), Wf[...],
                           preferred_element_type=jnp.float32) + bf_[...]


def _mw0_body(h_ref, We, mw_out):
    mw_out[...] = jnp.dot(_bf(h_ref[...]), We[...],
                          preferred_element_type=jnp.float32)


_gru_call = pl.pallas_call(
    _gru_body,
    out_shape=(jax.ShapeDtypeStruct((NP, H), jnp.float32),
               jax.ShapeDtypeStruct((NP, H), jnp.float32)))
_gru_final_call = pl.pallas_call(
    _gru_final_body, out_shape=jax.ShapeDtypeStruct((NP, H), jnp.float32))
_mw0_call = pl.pallas_call(
    _mw0_body, out_shape=jax.ShapeDtypeStruct((NP, H), jnp.float32))


def kernel(node_features, edge_index, final_molecule_indcs,
           W_edge, W_i, W_h, b_i, b_h, W_final, b_final):
    hp = jnp.zeros((NP, H), jnp.float32).at[:N].set(node_features)
    pad = EP - E
    src_p = jnp.concatenate(
        [edge_index[0], jnp.zeros((pad,), jnp.int32)])
    dst_p = jnp.concatenate(
        [edge_index[1], jnp.full((pad,), NP - 1, jnp.int32)])
    src4 = src_p.reshape(NW, NPASS, PCH, C)
    dst4 = dst_p.reshape(NW, NPASS, PCH, C)
    zero = jnp.zeros((C, H), jnp.float32)
    bi = b_i.reshape(1, 3 * H)
    bh = b_h.reshape(1, 3 * H)
    bf_ = b_final.reshape(1, H)
    We_b = _bf(W_edge)
    Wi_b = _bf(W_i)
    Wh_b = _bf(W_h)
    Wf_b = _bf(W_final)

    h = hp
    mw = _mw0_call(hp, We_b)
    for t in range(T):
        parts = _sc_segsum(mw, src4, dst4, zero)
        if t < T - 1:
            h, mw = _gru_call(parts, h, We_b, Wi_b, Wh_b, bi, bh)
        else:
            zfin = _gru_final_call(parts, h, Wi_b, Wh_b, bi, bh, Wf_b, bf_)
    return _sc_gather(zfin, final_molecule_indcs)


# speedup vs baseline: 3.5672x; 3.5672x over previous
import functools

import jax
import jax.numpy as jnp
from jax import lax
from jax.experimental import pallas as pl
from jax.experimental.pallas import tpu as pltpu
from jax.experimental.pallas import tpu_sc as plsc

N = 10000
NP = 10240
H = 128
E = 320000
B = 512
T = 4

NC = 2
NS = 16
NW = NC * NS
C = 80
PCH = 64
NPASS = 2
EWP = NPASS * PCH * C
EP = NW * EWP
RT = NP // NS
ZC = RT // C
BW = B // NW

_sc_mesh = plsc.VectorSubcoreMesh(
    core_axis_name="c", subcore_axis_name="s", num_cores=NC, num_subcores=NS)


@functools.partial(
    pl.kernel,
    out_type=jax.ShapeDtypeStruct((NC, NP, H), jnp.float32),
    mesh=_sc_mesh,
    scratch_types=[
        pltpu.VMEM((PCH, C), jnp.int32),
        pltpu.VMEM((PCH, C), jnp.int32),
        pltpu.VMEM((C, H), jnp.float32),
        pltpu.VMEM((C, H), jnp.float32),
        pltpu.VMEM_SHARED((NP, H), jnp.float32),
        pltpu.SemaphoreType.DMA,
        pltpu.SemaphoreType.DMA,
    ],
)
def _sc_segsum(mw_hbm, src_hbm, dst_hbm, zero_hbm, out_hbm,
               sidx, didx, rows0, rows1, acc, sem0, sem1):
    cid = lax.axis_index("c")
    sid = lax.axis_index("s")
    wid = sid * NC + cid
    pltpu.sync_copy(zero_hbm, rows0)
    for k in range(ZC):
        pltpu.sync_copy(rows0, acc.at[pl.ds(sid * RT + k * C, C)])
    plsc.subcore_barrier()

    for p in range(NPASS):
        pltpu.sync_copy(src_hbm.at[wid, p], sidx)
        pltpu.sync_copy(dst_hbm.at[wid, p], didx)
        pltpu.async_copy(mw_hbm.at[sidx.at[0]], rows0, sem0)

        @pl.loop(0, PCH - 2, step=2)
        def _chunk(j):
            pltpu.async_copy(mw_hbm.at[sidx.at[j + 1]], rows1, sem1)
            pltpu.make_async_copy(mw_hbm.at[sidx.at[j]], rows0, sem0).wait()
            pltpu.sync_copy(rows0, acc.at[didx.at[j]], add=True)
            pltpu.async_copy(mw_hbm.at[sidx.at[j + 2]], rows0, sem0)
            pltpu.make_async_copy(
                mw_hbm.at[sidx.at[j + 1]], rows1, sem1).wait()
            pltpu.sync_copy(rows1, acc.at[didx.at[j + 1]], add=True)

        pltpu.async_copy(mw_hbm.at[sidx.at[PCH - 1]], rows1, sem1)
        pltpu.make_async_copy(
            mw_hbm.at[sidx.at[PCH - 2]], rows0, sem0).wait()
        pltpu.sync_copy(rows0, acc.at[didx.at[PCH - 2]], add=True)
        pltpu.make_async_copy(
            mw_hbm.at[sidx.at[PCH - 1]], rows1, sem1).wait()
        pltpu.sync_copy(rows1, acc.at[didx.at[PCH - 1]], add=True)

    plsc.subcore_barrier()
    for k in range(ZC):
        pltpu.sync_copy(acc.at[pl.ds(sid * RT + k * C, C)], rows0)
        pltpu.sync_copy(rows0, out_hbm.at[cid, pl.ds(sid * RT + k * C, C)])


@functools.partial(
    pl.kernel,
    out_type=jax.ShapeDtypeStruct((B, H), jnp.float32),
    mesh=_sc_mesh,
    scratch_types=[
        pltpu.VMEM((BW,), jnp.int32),
        pltpu.VMEM((BW, H), jnp.float32),
        pltpu.SemaphoreType.DMA,
    ],
)
def _sc_gather(z_hbm, idx_hbm, out_hbm, idx_v, rows_v, sem):
    wid = lax.axis_index("s") * NC + lax.axis_index("c")
    base = wid * BW
    pltpu.sync_copy(idx_hbm.at[pl.ds(base, BW)], idx_v)
    pltpu.async_copy(z_hbm.at[idx_v], rows_v, sem).wait()
    pltpu.sync_copy(rows_v, out_hbm.at[pl.ds(base, BW)])


def _bf(x):
    return x.astype(jnp.bfloat16)


def _gru_math(p_ref, h_ref, Wi, Wh, bi, bh):
    h = h_ref[...]
    agg = p_ref[0] + p_ref[1]
    gi = jnp.dot(_bf(agg), Wi[...],
                 preferred_element_type=jnp.float32) + bi[...]
    gh = jnp.dot(_bf(h), Wh[...],
                 preferred_element_type=jnp.float32) + bh[...]
    r = jax.nn.sigmoid(gi[:, :H] + gh[:, :H])
    z = jax.nn.sigmoid(gi[:, H:2 * H] + gh[:, H:2 * H])
    n = jnp.tanh(gi[:, 2 * H:] + r * gh[:, 2 * H:])
    return (1.0 - z) * n + z * h


def _gru_body(p_ref, h_ref, We, Wi, Wh, bi, bh, h_out, mw_out):
    h_new = _gru_math(p_ref, h_ref, Wi, Wh, bi, bh)
    h_out[...] = h_new
    mw_out[...] = jnp.dot(_bf(h_new), We[...],
                          preferred_element_type=jnp.float32)


def _gru_final_body(p_ref, h_ref, Wi, Wh, bi, bh, Wf, bf_, out_ref):
    h_new = _gru_math(p_ref, h_ref, Wi, Wh, bi, bh)
    out_ref[...] = jnp.dot(_bf(h_new), Wf[...],
                           preferred_element_type=jnp.float32) + bf_[...]


def _mw0_body(h_ref, We, mw_out):
    mw_out[...] = jnp.dot(_bf(h_ref[...]), We[...],
                          preferred_element_type=jnp.float32)


_gru_call = pl.pallas_call(
    _gru_body,
    out_shape=(jax.ShapeDtypeStruct((NP, H), jnp.float32),
               jax.ShapeDtypeStruct((NP, H), jnp.float32)))
_gru_final_call = pl.pallas_call(
    _gru_final_body, out_shape=jax.ShapeDtypeStruct((NP, H), jnp.float32))
_mw0_call = pl.pallas_call(
    _mw0_body, out_shape=jax.ShapeDtypeStruct((NP, H), jnp.float32))


def kernel(node_features, edge_index, final_molecule_indcs,
           W_edge, W_i, W_h, b_i, b_h, W_final, b_final):
    hp = jnp.zeros((NP, H), jnp.float32).at[:N].set(node_features)
    pad = EP - E
    pad_dst = N + (jnp.arange(pad, dtype=jnp.int32) % (NP - N))
    pad_src = jnp.arange(pad, dtype=jnp.int32) % N
    src_p = jnp.concatenate([edge_index[0], pad_src])
    dst_p = jnp.concatenate([edge_index[1], pad_dst])
    src4 = src_p.reshape(NW, NPASS, PCH, C)
    dst4 = dst_p.reshape(NW, NPASS, PCH, C)
    zero = jnp.zeros((C, H), jnp.float32)
    bi = b_i.reshape(1, 3 * H)
    bh = b_h.reshape(1, 3 * H)
    bf_ = b_final.reshape(1, H)
    We_b = _bf(W_edge)
    Wi_b = _bf(W_i)
    Wh_b = _bf(W_h)
    Wf_b = _bf(W_final)

    h = hp
    mw = _mw0_call(hp, We_b)
    for t in range(T):
        parts = _sc_segsum(mw, src4, dst4, zero)
        if t < T - 1:
            h, mw = _gru_call(parts, h, We_b, Wi_b, Wh_b, bi, bh)
        else:
            zfin = _gru_final_call(parts, h, Wi_b, Wh_b, bi, bh, Wf_b, bf_)
    return _sc_gather(zfin, final_molecule_indcs)


# speedup vs baseline: 3.9077x; 1.0955x over previous
import functools

import jax
import jax.numpy as jnp
from jax import lax
from jax.experimental import pallas as pl
from jax.experimental.pallas import tpu as pltpu
from jax.experimental.pallas import tpu_sc as plsc

N = 10000
NP = 10240
H = 128
E = 320000
B = 512
T = 4

NC = 2
NS = 16
NW = NC * NS
C = 128
PCH = 40
NPASS = 2
EWP = NPASS * PCH * C
EP = NW * EWP
RT = NP // NS
ZC = RT // C
BW = B // NW

_sc_mesh = plsc.VectorSubcoreMesh(
    core_axis_name="c", subcore_axis_name="s", num_cores=NC, num_subcores=NS)


@functools.partial(
    pl.kernel,
    out_type=jax.ShapeDtypeStruct((NC, NP, H), jnp.float32),
    mesh=_sc_mesh,
    scratch_types=[
        pltpu.VMEM((PCH, C), jnp.int32),
        pltpu.VMEM((PCH, C), jnp.int32),
        pltpu.VMEM((C, H), jnp.float32),
        pltpu.VMEM((C, H), jnp.float32),
        pltpu.VMEM_SHARED((NP, H), jnp.float32),
        pltpu.SemaphoreType.DMA,
        pltpu.SemaphoreType.DMA,
    ],
)
def _sc_segsum(mw_hbm, src_hbm, dst_hbm, zero_hbm, out_hbm,
               sidx, didx, rows0, rows1, acc, sem0, sem1):
    cid = lax.axis_index("c")
    sid = lax.axis_index("s")
    wid = sid * NC + cid
    pltpu.sync_copy(zero_hbm, rows0)
    for k in range(ZC):
        pltpu.sync_copy(rows0, acc.at[pl.ds(sid * RT + k * C, C)])
    plsc.subcore_barrier()

    for p in range(NPASS):
        pltpu.sync_copy(src_hbm.at[wid, p], sidx)
        pltpu.sync_copy(dst_hbm.at[wid, p], didx)
        pltpu.async_copy(mw_hbm.at[sidx.at[0]], rows0, sem0)

        @pl.loop(0, PCH - 2, step=2)
        def _chunk(j):
            pltpu.async_copy(mw_hbm.at[sidx.at[j + 1]], rows1, sem1)
            pltpu.make_async_copy(mw_hbm.at[sidx.at[j]], rows0, sem0).wait()
            pltpu.sync_copy(rows0, acc.at[didx.at[j]], add=True)
            pltpu.async_copy(mw_hbm.at[sidx.at[j + 2]], rows0, sem0)
            pltpu.make_async_copy(
                mw_hbm.at[sidx.at[j + 1]], rows1, sem1).wait()
            pltpu.sync_copy(rows1, acc.at[didx.at[j + 1]], add=True)

        pltpu.async_copy(mw_hbm.at[sidx.at[PCH - 1]], rows1, sem1)
        pltpu.make_async_copy(
            mw_hbm.at[sidx.at[PCH - 2]], rows0, sem0).wait()
        pltpu.sync_copy(rows0, acc.at[didx.at[PCH - 2]], add=True)
        pltpu.make_async_copy(
            mw_hbm.at[sidx.at[PCH - 1]], rows1, sem1).wait()
        pltpu.sync_copy(rows1, acc.at[didx.at[PCH - 1]], add=True)

    plsc.subcore_barrier()
    for k in range(ZC):
        pltpu.sync_copy(acc.at[pl.ds(sid * RT + k * C, C)], rows0)
        pltpu.sync_copy(rows0, out_hbm.at[cid, pl.ds(sid * RT + k * C, C)])


@functools.partial(
    pl.kernel,
    out_type=jax.ShapeDtypeStruct((B, H), jnp.float32),
    mesh=_sc_mesh,
    scratch_types=[
        pltpu.VMEM((BW,), jnp.int32),
        pltpu.VMEM((BW, H), jnp.float32),
        pltpu.SemaphoreType.DMA,
    ],
)
def _sc_gather(z_hbm, idx_hbm, out_hbm, idx_v, rows_v, sem):
    wid = lax.axis_index("s") * NC + lax.axis_index("c")
    base = wid * BW
    pltpu.sync_copy(idx_hbm.at[pl.ds(base, BW)], idx_v)
    pltpu.async_copy(z_hbm.at[idx_v], rows_v, sem).wait()
    pltpu.sync_copy(rows_v, out_hbm.at[pl.ds(base, BW)])


def _bf(x):
    return x.astype(jnp.bfloat16)


def _gru_math(p_ref, h_ref, Wi, Wh, bi, bh):
    h = h_ref[...]
    agg = p_ref[0] + p_ref[1]
    gi = jnp.dot(_bf(agg), Wi[...],
                 preferred_element_type=jnp.float32) + bi[...]
    gh = jnp.dot(_bf(h), Wh[...],
                 preferred_element_type=jnp.float32) + bh[...]
    r = jax.nn.sigmoid(gi[:, :H] + gh[:, :H])
    z = jax.nn.sigmoid(gi[:, H:2 * H] + gh[:, H:2 * H])
    n = jnp.tanh(gi[:, 2 * H:] + r * gh[:, 2 * H:])
    return (1.0 - z) * n + z * h


def _gru_body(p_ref, h_ref, We, Wi, Wh, bi, bh, h_out, mw_out):
    h_new = _gru_math(p_ref, h_ref, Wi, Wh, bi, bh)
    h_out[...] = h_new
    mw_out[...] = jnp.dot(_bf(h_new), We[...],
                          preferred_element_type=jnp.float32)


def _gru_final_body(p_ref, h_ref, Wi, Wh, bi, bh, Wf, bf_, out_ref):
    h_new = _gru_math(p_ref, h_ref, Wi, Wh, bi, bh)
    out_ref[...] = jnp.dot(_bf(h_new), Wf[...],
                           preferred_element_type=jnp.float32) + bf_[...]


def _mw0_body(h_ref, We, mw_out):
    mw_out[...] = jnp.dot(_bf(h_ref[...]), We[...],
                          preferred_element_type=jnp.float32)


_gru_call = pl.pallas_call(
    _gru_body,
    out_shape=(jax.ShapeDtypeStruct((NP, H), jnp.float32),
               jax.ShapeDtypeStruct((NP, H), jnp.float32)))
_gru_final_call = pl.pallas_call(
    _gru_final_body, out_shape=jax.ShapeDtypeStruct((NP, H), jnp.float32))
_mw0_call = pl.pallas_call(
    _mw0_body, out_shape=jax.ShapeDtypeStruct((NP, H), jnp.float32))


def kernel(node_features, edge_index, final_molecule_indcs,
           W_edge, W_i, W_h, b_i, b_h, W_final, b_final):
    hp = jnp.zeros((NP, H), jnp.float32).at[:N].set(node_features)
    pad = EP - E
    pad_dst = N + (jnp.arange(pad, dtype=jnp.int32) % (NP - N))
    pad_src = jnp.arange(pad, dtype=jnp.int32) % N
    src_p = jnp.concatenate([edge_index[0], pad_src])
    dst_p = jnp.concatenate([edge_index[1], pad_dst])
    src4 = src_p.reshape(NW, NPASS, PCH, C)
    dst4 = dst_p.reshape(NW, NPASS, PCH, C)
    zero = jnp.zeros((C, H), jnp.float32)
    bi = b_i.reshape(1, 3 * H)
    bh = b_h.reshape(1, 3 * H)
    bf_ = b_final.reshape(1, H)
    We_b = _bf(W_edge)
    Wi_b = _bf(W_i)
    Wh_b = _bf(W_h)
    Wf_b = _bf(W_final)

    h = hp
    mw = _mw0_call(hp, We_b)
    for t in range(T):
        parts = _sc_segsum(mw, src4, dst4, zero)
        if t < T - 1:
            h, mw = _gru_call(parts, h, We_b, Wi_b, Wh_b, bi, bh)
        else:
            zfin = _gru_final_call(parts, h, Wi_b, Wh_b, bi, bh, Wf_b, bf_)
    return _sc_gather(zfin, final_molecule_indcs)
